# baseline (speedup 1.0000x reference)
import functools

import jax
import jax.numpy as jnp
from jax import lax
from jax.experimental import pallas as pl
from jax.experimental.pallas import tpu as pltpu
from jax.experimental.pallas import tpu_sc as plsc

N_NODE = 10000
EMB = 128
N_EDGES = 320000
LAYERS = 3

E = 80
PK = 2 * E
EW = 16 * E
NCHUNKS = N_EDGES // E
NW = 32
KU = NCHUNKS // NW
ROWCH = 80
NROWCH = N_NODE // ROWCH
KROW = (NROWCH + 15) // 16
KMRG = (NROWCH + NW - 1) // NW

_mesh = plsc.VectorSubcoreMesh(core_axis_name="c", subcore_axis_name="s")


@functools.partial(
    pl.kernel,
    out_type=[
        jax.ShapeDtypeStruct((N_NODE, EMB), jnp.float32),
        jax.ShapeDtypeStruct((N_NODE, EMB), jnp.float32),
        jax.ShapeDtypeStruct((N_NODE, EMB), jnp.float32),
        jax.ShapeDtypeStruct((N_NODE, EMB), jnp.float32),
    ],
    mesh=_mesh,
    scratch_types=[
        pltpu.VMEM_SHARED((N_NODE, EMB), jnp.float32),
        pltpu.VMEM((PK,), jnp.int32),
        pltpu.VMEM((PK,), jnp.int32),
        pltpu.VMEM((PK,), jnp.int32),
        pltpu.VMEM((PK,), jnp.int32),
        pltpu.VMEM((EW,), jnp.float32),
        pltpu.VMEM((EW,), jnp.float32),
        pltpu.VMEM((EW,), jnp.float32),
        pltpu.VMEM((EW,), jnp.float32),
        pltpu.VMEM((E,), jnp.int32),
        pltpu.VMEM((E,), jnp.int32),
        pltpu.VMEM((E,), jnp.int32),
        pltpu.VMEM((E,), jnp.int32),
        pltpu.VMEM((E, EMB), jnp.float32),
        pltpu.VMEM((E, EMB), jnp.float32),
        pltpu.VMEM((E, EMB), jnp.float32),
        pltpu.VMEM((E, EMB), jnp.float32),
        pltpu.SemaphoreType.DMA,
        pltpu.SemaphoreType.DMA,
        pltpu.SemaphoreType.DMA,
        pltpu.SemaphoreType.DMA,
        pltpu.SemaphoreType.DMA,
        pltpu.SemaphoreType.DMA,
        pltpu.SemaphoreType.DMA,
        pltpu.SemaphoreType.DMA,
        pltpu.SemaphoreType.DMA,
        pltpu.SemaphoreType.DMA,
        pltpu.SemaphoreType.DMA,
        pltpu.SemaphoreType.DMA,
        pltpu.SemaphoreType.REGULAR,
    ],
)
def _hconv(emb_hbm, edata_hbm, w16_hbm, out_hbm, pe0_hbm, pe1_hbm, xs_hbm,
           acc_sh, eb0, eb1, eb2, eb3, vb0, vb1, vb2, vb3,
           db0, db1, db2, db3, rw0, rw1, rw2, rw3,
           isem0, isem1, isem2, isem3, gsem0, gsem1, gsem2, gsem3,
           ssem0, ssem1, ssem2, ssem3, xsem):
    cid = lax.axis_index("c")
    sid = lax.axis_index("s")
    wid = sid * 2 + cid

    eb = [eb0, eb1, eb2, eb3]
    vb = [vb0, vb1, vb2, vb3]
    db = [db0, db1, db2, db3]
    rw = [rw0, rw1, rw2, rw3]
    isem = [isem0, isem1, isem2, isem3]
    gsem = [gsem0, gsem1, gsem2, gsem3]
    ssem = [ssem0, ssem1, ssem2, ssem3]

    def zero_rw0():
        def _z(r, carry):
            for i in range(8):
                rw0[r, pl.ds(i * 16, 16)] = jnp.zeros((16,), jnp.float32)
            return carry
        lax.fori_loop(0, ROWCH, _z, 0)

    def add_into(dst, src):
        def _a(r, carry):
            for i in range(8):
                sl = pl.ds(i * 16, 16)
                dst[r, sl] = dst[r, sl] + src[r, sl]
            return carry
        lax.fori_loop(0, ROWCH, _a, 0)

    def scale_by(dst, f):
        def _m(r, carry):
            for i in range(8):
                sl = pl.ds(i * 16, 16)
                dst[r, sl] = dst[r, sl] * f
            return carry
        lax.fori_loop(0, ROWCH, _m, 0)

    def gbarrier():
        plsc.subcore_barrier()

        @pl.when(sid == 0)
        def _():
            pltpu.semaphore_signal(xsem, 1, device_id={"c": 1 - cid, "s": 0})
            pltpu.semaphore_wait(xsem, 1)

        plsc.subcore_barrier()

    def start_idx(c, p):
        j = wid + c * NW
        pltpu.async_copy(edata_hbm.at[pl.ds(j * PK, PK)], eb[p], isem[p])
        pltpu.async_copy(w16_hbm.at[pl.ds(j * EW, EW)], vb[p], isem[p])

    def wait_idx(p):
        pltpu.make_async_copy(edata_hbm.at[pl.ds(0, PK)], eb[p], isem[p]).wait()
        pltpu.make_async_copy(w16_hbm.at[pl.ds(0, EW)], vb[p], isem[p]).wait()

    def start_gather(p, src):
        pltpu.async_copy(src.at[eb[p].at[pl.ds(0, E)]], rw[p], gsem[p])

    def wait_gather(p, src):
        pltpu.make_async_copy(src.at[eb[p].at[pl.ds(0, E)]],
                              rw[p], gsem[p]).wait()

    def start_scatter(p):
        pltpu.async_copy(rw[p], acc_sh.at[db[p]], ssem[p], add=True)

    def wait_scatter(p):
        pltpu.make_async_copy(rw[p], acc_sh.at[db[p]], ssem[p]).wait()

    def scale_and_dst(p):
        ebp, vbp, rwp, dbp = eb[p], vb[p], rw[p], db[p]

        def _scale(r, carry):
            wr = vbp[pl.ds(r * 16, 16)]
            for i in range(8):
                sl = pl.ds(i * 16, 16)
                rwp[r, sl] = rwp[r, sl] * wr
            return carry
        lax.fori_loop(0, E, _scale, 0)

        def _dst(g, carry):
            sl = pl.ds(g * 16, 16)
            dbp[sl] = ebp[pl.ds(E + g * 16, 16)]
            return carry
        lax.fori_loop(0, E // 16, _dst, 0)

    def run_layer(src):
        def proc(c, p, gather_ahead, idx_ahead, wait_prev_scatter):
            p2 = (p + 2) % 4
            wait_gather(p, src)
            if gather_ahead:
                if wait_prev_scatter:
                    wait_scatter(p2)
                wait_idx(p2)
                start_gather(p2, src)
            scale_and_dst(p)
            start_scatter(p)
            if idx_ahead:
                start_idx(c + 4, p)

        for c0 in range(4):
            start_idx(c0, c0)
        wait_idx(0)
        start_gather(0, src)
        wait_idx(1)
        start_gather(1, src)
        proc(0, 0, True, True, False)
        proc(1, 1, True, True, False)

        nq = (KU - 6) // 4
        steady_end = 2 + 4 * nq

        def _quad(k, carry):
            c = 4 * k + 2
            proc(c, 2, True, True, True)
            proc(c + 1, 3, True, True, True)
            proc(c + 2, 0, True, True, True)
            proc(c + 3, 1, True, True, True)
            return carry
        lax.fori_loop(0, nq, _quad, 0)

        for c in range(steady_end, KU - 4):
            proc(c, c % 4, True, True, True)
        for c in range(KU - 4, KU - 2):
            proc(c, c % 4, True, False, True)
        for c in range(KU - 2, KU):
            proc(c, c % 4, False, False, False)
        for c in range(KU - 4, KU):
            wait_scatter(c % 4)

        plsc.subcore_barrier()

    def export_and_rezero(rezero):
        if rezero:
            zero_rw0()
        for k in range(KROW):
            c = k * 16 + sid
            sl = pl.ds(c * ROWCH, ROWCH)

            @pl.when(jnp.logical_and(c < NROWCH, cid == 0))
            def _():
                pltpu.sync_copy(acc_sh.at[sl], pe0_hbm.at[sl])

            @pl.when(jnp.logical_and(c < NROWCH, cid == 1))
            def _():
                pltpu.sync_copy(acc_sh.at[sl], pe1_hbm.at[sl])

            if rezero:
                @pl.when(c < NROWCH)
                def _():
                    pltpu.sync_copy(rw0.at[pl.ds(0, ROWCH)], acc_sh.at[sl])

    def merge(first):
        for k in range(KMRG):
            c = k * NW + wid

            @pl.when(c < NROWCH)
            def _():
                sl = pl.ds(c * ROWCH, ROWCH)
                pltpu.sync_copy(pe0_hbm.at[sl], rw0)
                pltpu.sync_copy(pe1_hbm.at[sl], rw1)
                add_into(rw0, rw1)
                pltpu.sync_copy(rw0, xs_hbm.at[sl])
                if first:
                    pltpu.sync_copy(emb_hbm.at[sl], rw1)
                else:
                    pltpu.sync_copy(out_hbm.at[sl], rw1)
                add_into(rw0, rw1)
                pltpu.sync_copy(rw0, out_hbm.at[sl])

    def final_phase():
        for k in range(KMRG):
            c = k * NW + wid

            @pl.when(c < NROWCH)
            def _():
                sl = pl.ds(c * ROWCH, ROWCH)
                pltpu.sync_copy(pe0_hbm.at[sl], rw0)
                pltpu.sync_copy(pe1_hbm.at[sl], rw1)
                add_into(rw0, rw1)
                pltpu.sync_copy(out_hbm.at[sl], rw1)
                add_into(rw0, rw1)
                scale_by(rw0, 0.25)
                pltpu.sync_copy(rw0, out_hbm.at[sl])

    zero_rw0()
    for k in range(KROW):
        c = k * 16 + sid

        @pl.when(c < NROWCH)
        def _():
            pltpu.sync_copy(rw0.at[pl.ds(0, ROWCH)],
                            acc_sh.at[pl.ds(c * ROWCH, ROWCH)])
    plsc.subcore_barrier()

    run_layer(emb_hbm)
    export_and_rezero(rezero=True)
    gbarrier()
    merge(first=True)
    gbarrier()

    run_layer(xs_hbm)
    export_and_rezero(rezero=True)
    gbarrier()
    merge(first=False)
    gbarrier()

    run_layer(xs_hbm)
    export_and_rezero(rezero=False)
    gbarrier()
    final_phase()


@jax.jit
def kernel(embedding, edge_index, edge_weight):
    src = edge_index[1].reshape(NCHUNKS, E)
    dst = edge_index[0].reshape(NCHUNKS, E)
    edata = jnp.stack([src, dst], axis=1).reshape(NCHUNKS * PK)
    w16 = jnp.broadcast_to(edge_weight[:, None],
                           (N_EDGES, 16)).reshape(N_EDGES * 16)
    out, _, _, _ = _hconv(embedding, edata, w16)
    return out


# speedup vs baseline: 1.0482x; 1.0482x over previous
import functools

import jax
import jax.numpy as jnp
from jax import lax
from jax.experimental import pallas as pl
from jax.experimental.pallas import tpu as pltpu
from jax.experimental.pallas import tpu_sc as plsc

N_NODE = 10000
EMB = 128
N_EDGES = 320000
LAYERS = 3

E = 80
PK = 2 * E
EW = 16 * E
NCHUNKS = N_EDGES // E
NW = 32
KU = NCHUNKS // NW
ROWCH = 80
NROWCH = N_NODE // ROWCH
KROW = (NROWCH + 15) // 16
KMRG = (NROWCH + NW - 1) // NW

_mesh = plsc.VectorSubcoreMesh(core_axis_name="c", subcore_axis_name="s")


@functools.partial(
    pl.kernel,
    out_type=[
        jax.ShapeDtypeStruct((N_NODE, EMB), jnp.float32),
        jax.ShapeDtypeStruct((N_NODE, EMB), jnp.float32),
        jax.ShapeDtypeStruct((N_NODE, EMB), jnp.float32),
        jax.ShapeDtypeStruct((N_NODE, EMB), jnp.float32),
    ],
    mesh=_mesh,
    scratch_types=[
        pltpu.VMEM_SHARED((N_NODE, EMB), jnp.float32),
        pltpu.VMEM((PK,), jnp.int32),
        pltpu.VMEM((PK,), jnp.int32),
        pltpu.VMEM((PK,), jnp.int32),
        pltpu.VMEM((PK,), jnp.int32),
        pltpu.VMEM((EW,), jnp.float32),
        pltpu.VMEM((EW,), jnp.float32),
        pltpu.VMEM((EW,), jnp.float32),
        pltpu.VMEM((EW,), jnp.float32),
        pltpu.VMEM((E,), jnp.int32),
        pltpu.VMEM((E,), jnp.int32),
        pltpu.VMEM((E,), jnp.int32),
        pltpu.VMEM((E,), jnp.int32),
        pltpu.VMEM((E, EMB), jnp.float32),
        pltpu.VMEM((E, EMB), jnp.float32),
        pltpu.VMEM((E, EMB), jnp.float32),
        pltpu.VMEM((E, EMB), jnp.float32),
        pltpu.SemaphoreType.DMA,
        pltpu.SemaphoreType.DMA,
        pltpu.SemaphoreType.DMA,
        pltpu.SemaphoreType.DMA,
        pltpu.SemaphoreType.DMA,
        pltpu.SemaphoreType.DMA,
        pltpu.SemaphoreType.DMA,
        pltpu.SemaphoreType.DMA,
        pltpu.SemaphoreType.DMA,
        pltpu.SemaphoreType.DMA,
        pltpu.SemaphoreType.DMA,
        pltpu.SemaphoreType.DMA,
        pltpu.SemaphoreType.REGULAR,
    ],
)
def _hconv(emb_hbm, edata_hbm, w16_hbm, out_hbm, pe0_hbm, pe1_hbm, xs_hbm,
           acc_sh, eb0, eb1, eb2, eb3, vb0, vb1, vb2, vb3,
           db0, db1, db2, db3, rw0, rw1, rw2, rw3,
           isem0, isem1, isem2, isem3, gsem0, gsem1, gsem2, gsem3,
           ssem0, ssem1, ssem2, ssem3, xsem):
    cid = lax.axis_index("c")
    sid = lax.axis_index("s")
    wid = sid * 2 + cid

    eb = [eb0, eb1, eb2, eb3]
    vb = [vb0, vb1, vb2, vb3]
    db = [db0, db1, db2, db3]
    rw = [rw0, rw1, rw2, rw3]
    isem = [isem0, isem1, isem2, isem3]
    gsem = [gsem0, gsem1, gsem2, gsem3]
    ssem = [ssem0, ssem1, ssem2, ssem3]

    def zero_rw0():
        def _z(r, carry):
            for i in range(8):
                rw0[r, pl.ds(i * 16, 16)] = jnp.zeros((16,), jnp.float32)
            return carry
        lax.fori_loop(0, ROWCH, _z, 0)

    def add_into(dst, src):
        def _a(r, carry):
            for i in range(8):
                sl = pl.ds(i * 16, 16)
                dst[r, sl] = dst[r, sl] + src[r, sl]
            return carry
        lax.fori_loop(0, ROWCH, _a, 0)

    def sub_from(dst, src):
        def _s(r, carry):
            for i in range(8):
                sl = pl.ds(i * 16, 16)
                dst[r, sl] = dst[r, sl] - src[r, sl]
            return carry
        lax.fori_loop(0, ROWCH, _s, 0)

    def scale_by(dst, f):
        def _m(r, carry):
            for i in range(8):
                sl = pl.ds(i * 16, 16)
                dst[r, sl] = dst[r, sl] * f
            return carry
        lax.fori_loop(0, ROWCH, _m, 0)

    def gbarrier():
        plsc.subcore_barrier()

        @pl.when(sid == 0)
        def _():
            pltpu.semaphore_signal(xsem, 1, device_id={"c": 1 - cid, "s": 0})
            pltpu.semaphore_wait(xsem, 1)

        plsc.subcore_barrier()

    def start_idx(c, p):
        j = wid + c * NW
        pltpu.async_copy(edata_hbm.at[pl.ds(j * PK, PK)], eb[p], isem[p])
        pltpu.async_copy(w16_hbm.at[pl.ds(j * EW, EW)], vb[p], isem[p])

    def wait_idx(p):
        pltpu.make_async_copy(edata_hbm.at[pl.ds(0, PK)], eb[p], isem[p]).wait()
        pltpu.make_async_copy(w16_hbm.at[pl.ds(0, EW)], vb[p], isem[p]).wait()

    def start_gather(p, src):
        pltpu.async_copy(src.at[eb[p].at[pl.ds(0, E)]], rw[p], gsem[p])

    def wait_gather(p, src):
        pltpu.make_async_copy(src.at[eb[p].at[pl.ds(0, E)]],
                              rw[p], gsem[p]).wait()

    def start_scatter(p):
        pltpu.async_copy(rw[p], acc_sh.at[db[p]], ssem[p], add=True)

    def wait_scatter(p):
        pltpu.make_async_copy(rw[p], acc_sh.at[db[p]], ssem[p]).wait()

    def scale_and_dst(p):
        ebp, vbp, rwp, dbp = eb[p], vb[p], rw[p], db[p]

        def _scale(q, carry):
            for rr in range(4):
                r = q * 4 + rr
                wr = vbp[pl.ds(r * 16, 16)]
                for i in range(8):
                    sl = pl.ds(i * 16, 16)
                    rwp[r, sl] = rwp[r, sl] * wr
            return carry
        lax.fori_loop(0, E // 4, _scale, 0)

        def _dst(g, carry):
            sl = pl.ds(g * 16, 16)
            dbp[sl] = ebp[pl.ds(E + g * 16, 16)]
            return carry
        lax.fori_loop(0, E // 16, _dst, 0)

    def run_layer(src):
        def proc(c, p, gather_ahead, idx_ahead, wait_prev_scatter):
            p2 = (p + 2) % 4
            wait_gather(p, src)
            if gather_ahead:
                if wait_prev_scatter:
                    wait_scatter(p2)
                wait_idx(p2)
                start_gather(p2, src)
            scale_and_dst(p)
            start_scatter(p)
            if idx_ahead:
                start_idx(c + 4, p)

        for c0 in range(4):
            start_idx(c0, c0)
        wait_idx(0)
        start_gather(0, src)
        wait_idx(1)
        start_gather(1, src)
        proc(0, 0, True, True, False)
        proc(1, 1, True, True, False)

        nq = (KU - 6) // 4
        steady_end = 2 + 4 * nq

        def _quad(k, carry):
            c = 4 * k + 2
            proc(c, 2, True, True, True)
            proc(c + 1, 3, True, True, True)
            proc(c + 2, 0, True, True, True)
            proc(c + 3, 1, True, True, True)
            return carry
        lax.fori_loop(0, nq, _quad, 0)

        for c in range(steady_end, KU - 4):
            proc(c, c % 4, True, True, True)
        for c in range(KU - 4, KU - 2):
            proc(c, c % 4, True, False, True)
        for c in range(KU - 2, KU):
            proc(c, c % 4, False, False, False)
        for c in range(KU - 4, KU):
            wait_scatter(c % 4)

        plsc.subcore_barrier()

    def export():
        for k in range(KROW):
            c = k * 16 + sid
            sl = pl.ds(c * ROWCH, ROWCH)

            @pl.when(jnp.logical_and(c < NROWCH, cid == 0))
            def _():
                pltpu.sync_copy(acc_sh.at[sl], pe0_hbm.at[sl])

            @pl.when(jnp.logical_and(c < NROWCH, cid == 1))
            def _():
                pltpu.sync_copy(acc_sh.at[sl], pe1_hbm.at[sl])

    def merge(first):
        for k in range(KMRG):
            c = k * NW + wid

            @pl.when(c < NROWCH)
            def _():
                sl = pl.ds(c * ROWCH, ROWCH)
                cp0 = pltpu.async_copy(pe0_hbm.at[sl], rw0, gsem0)
                cp1 = pltpu.async_copy(pe1_hbm.at[sl], rw1, gsem1)
                if not first:
                    cp2 = pltpu.async_copy(xs_hbm.at[sl], rw2, gsem2)
                cp0.wait()
                cp1.wait()
                add_into(rw0, rw1)
                if not first:
                    cp2.wait()
                    sub_from(rw0, rw2)
                pltpu.sync_copy(rw0, xs_hbm.at[sl])

    def final_phase():
        for k in range(KMRG):
            c = k * NW + wid

            @pl.when(c < NROWCH)
            def _():
                sl = pl.ds(c * ROWCH, ROWCH)
                cp0 = pltpu.async_copy(pe0_hbm.at[sl], rw0, gsem0)
                cp1 = pltpu.async_copy(pe1_hbm.at[sl], rw1, gsem1)
                cp2 = pltpu.async_copy(emb_hbm.at[sl], rw2, gsem2)
                cp0.wait()
                cp1.wait()
                add_into(rw0, rw1)
                cp2.wait()
                add_into(rw0, rw2)
                scale_by(rw0, 0.25)
                pltpu.sync_copy(rw0, out_hbm.at[sl])

    zero_rw0()
    for k in range(KROW):
        c = k * 16 + sid

        @pl.when(c < NROWCH)
        def _():
            pltpu.sync_copy(rw0.at[pl.ds(0, ROWCH)],
                            acc_sh.at[pl.ds(c * ROWCH, ROWCH)])
    plsc.subcore_barrier()

    run_layer(emb_hbm)
    export()
    gbarrier()
    merge(first=True)
    gbarrier()

    run_layer(xs_hbm)
    export()
    gbarrier()
    merge(first=False)
    gbarrier()

    run_layer(xs_hbm)
    export()
    gbarrier()
    final_phase()


@jax.jit
def kernel(embedding, edge_index, edge_weight):
    src = edge_index[1].reshape(NCHUNKS, E)
    dst = edge_index[0].reshape(NCHUNKS, E)
    edata = jnp.stack([src, dst], axis=1).reshape(NCHUNKS * PK)
    w16 = jnp.broadcast_to(edge_weight[:, None],
                           (N_EDGES, 16)).reshape(N_EDGES * 16)
    out, _, _, _ = _hconv(embedding, edata, w16)
    return out
